# Initial kernel scaffold; baseline (speedup 1.0000x reference)
import jax
import jax.numpy as jnp
from jax.experimental import pallas as pl


def kernel(h, edge_index, weight, bias):
    raise NotImplementedError("write your pallas kernel here")


# speedup vs baseline: 7.6548x; 7.6548x over previous
import functools

import jax
import jax.numpy as jnp
from jax import lax
from jax.experimental import pallas as pl
from jax.experimental.pallas import tpu as pltpu
from jax.experimental.pallas import tpu_sc as plsc

NC = 2
NS = 16
NW = NC * NS
CW = 125


def _seg_body(nchunks, rows_per_tile, x_hbm, src_hbm, dst_hbm, zeros_hbm,
              out_hbm, srcidx_v, dstidx_v, rows_v, acc_sh, sem):
    c = lax.axis_index("c")
    s = lax.axis_index("s")
    w = s * NC + c

    pltpu.sync_copy(zeros_hbm, acc_sh.at[pl.ds(s * rows_per_tile, rows_per_tile)])
    pltpu.sync_copy(src_hbm.at[w], srcidx_v)
    pltpu.sync_copy(dst_hbm.at[w], dstidx_v)
    plsc.subcore_barrier()

    def step(j, carry):
        pltpu.async_copy(x_hbm.at[srcidx_v.at[j]], rows_v, sem).wait()
        pltpu.sync_copy(rows_v, acc_sh.at[dstidx_v.at[j]], add=True)
        return carry

    lax.fori_loop(0, nchunks, step, 0)
    plsc.subcore_barrier()
    pltpu.sync_copy(acc_sh.at[pl.ds(s * rows_per_tile, rows_per_tile)],
                    out_hbm.at[c, pl.ds(s * rows_per_tile, rows_per_tile)])


def _segment_sum_sc(x, src_r, dst_r, zeros_tile, n_pad):
    n, f = x.shape
    nchunks = src_r.shape[1]
    rows_per_tile = n_pad // NS
    mesh = plsc.VectorSubcoreMesh(core_axis_name="c", subcore_axis_name="s")
    body = functools.partial(_seg_body, nchunks, rows_per_tile)
    return pl.kernel(
        body,
        out_type=jax.ShapeDtypeStruct((NC, n_pad, f), jnp.float32),
        mesh=mesh,
        scratch_types=[
            pltpu.VMEM((nchunks, CW), jnp.int32),
            pltpu.VMEM((nchunks, CW), jnp.int32),
            pltpu.VMEM((CW, f), jnp.float32),
            pltpu.VMEM_SHARED((n_pad, f), jnp.float32),
            pltpu.SemaphoreType.DMA,
        ],
    )(x, src_r, dst_r, zeros_tile)


def _combine_body(scale, p_ref, o_ref):
    o_ref[...] = scale * (p_ref[0] + p_ref[1])


def _final_body(scale, h0_ref, h1_ref, q_ref, w_ref, b_ref, o_ref):
    h2 = scale * (q_ref[0] + q_ref[1])
    acc = jnp.dot(h0_ref[...], w_ref[0], preferred_element_type=jnp.float32)
    acc += jnp.dot(h1_ref[...], w_ref[1], preferred_element_type=jnp.float32)
    acc += jnp.dot(h2, w_ref[2], preferred_element_type=jnp.float32)
    o_ref[...] = acc + b_ref[...]


def kernel(h, edge_index, weight, bias):
    n, f = h.shape
    e = edge_index.shape[1]
    k = weight.shape[2]
    assert k == 3 and e % (NW * CW) == 0
    scale = 1.0 / n
    nchunks = e // (NW * CW)
    n_pad = ((n + NS * 8 - 1) // (NS * 8)) * NS * 8
    src_r = edge_index[0].reshape(NW, nchunks, CW)
    dst_r = edge_index[1].reshape(NW, nchunks, CW)
    zeros_tile = jnp.zeros((n_pad // NS, f), jnp.float32)
    wt = jnp.transpose(weight, (2, 0, 1))

    p1 = _segment_sum_sc(h, src_r, dst_r, zeros_tile, n_pad)

    bn = 1000
    grid = n // bn
    h1 = pl.pallas_call(
        functools.partial(_combine_body, scale),
        out_shape=jax.ShapeDtypeStruct((n, f), jnp.float32),
        grid=(grid,),
        in_specs=[pl.BlockSpec((NC, bn, f), lambda i: (0, i, 0))],
        out_specs=pl.BlockSpec((bn, f), lambda i: (i, 0)),
    )(p1)

    p2 = _segment_sum_sc(h1, src_r, dst_r, zeros_tile, n_pad)

    out = pl.pallas_call(
        functools.partial(_final_body, scale),
        out_shape=jax.ShapeDtypeStruct((n, f), jnp.float32),
        grid=(grid,),
        in_specs=[
            pl.BlockSpec((bn, f), lambda i: (i, 0)),
            pl.BlockSpec((bn, f), lambda i: (i, 0)),
            pl.BlockSpec((NC, bn, f), lambda i: (0, i, 0)),
            pl.BlockSpec((k, f, f), lambda i: (0, 0, 0)),
            pl.BlockSpec((1, f), lambda i: (0, 0)),
        ],
        out_specs=pl.BlockSpec((bn, f), lambda i: (i, 0)),
    )(h, h1, p2, wt, bias.reshape(1, f))
    return out


# speedup vs baseline: 9.4032x; 1.2284x over previous
import functools

import jax
import jax.numpy as jnp
from jax import lax
from jax.experimental import pallas as pl
from jax.experimental.pallas import tpu as pltpu
from jax.experimental.pallas import tpu_sc as plsc

NC = 2
NS = 16
CW = 125
NBUF = 2


def _chunk_loop(nchunks, src_sh, acc_sh, sidx, didx, rows, sems):
    for b in range(NBUF):
        pltpu.async_copy(src_sh.at[sidx.at[b]], rows[b], sems[b])

    def outer(i2, carry):
        for b in range(NBUF):
            j = i2 * NBUF + b
            pltpu.make_async_copy(src_sh.at[sidx.at[j]], rows[b], sems[b]).wait()
            pltpu.sync_copy(rows[b], acc_sh.at[didx.at[j]], add=True)
            nj = j + NBUF

            @pl.when(nj < nchunks)
            def _():
                pltpu.async_copy(src_sh.at[sidx.at[nj]], rows[b], sems[b])
        return carry

    lax.fori_loop(0, nchunks // NBUF, outer, 0)


def _gcn_body(nchunks, rpt, fc, x_hbm, src_hbm, dst_hbm, zeros_hbm,
              a1_hbm, a2_hbm, sidx, didx, rows0, rows1, acc,
              sem0, sem1):
    c = lax.axis_index("c")
    s = lax.axis_index("s")
    row0 = s * rpt
    pltpu.sync_copy(zeros_hbm, acc.at[pl.ds(row0, rpt)])
    pltpu.sync_copy(src_hbm.at[s], sidx)
    pltpu.sync_copy(dst_hbm.at[s], didx)
    plsc.subcore_barrier()

    rows = (rows0, rows1)
    sems = (sem0, sem1)
    _chunk_loop(nchunks, x_hbm.at[c], acc, sidx, didx, rows, sems)
    plsc.subcore_barrier()
    pltpu.sync_copy(acc.at[pl.ds(row0, rpt)], a1_hbm.at[c, pl.ds(row0, rpt)])
    plsc.subcore_barrier()
    pltpu.sync_copy(zeros_hbm, acc.at[pl.ds(row0, rpt)])
    plsc.subcore_barrier()
    _chunk_loop(nchunks, a1_hbm.at[c], acc, sidx, didx, rows, sems)
    plsc.subcore_barrier()
    pltpu.sync_copy(acc.at[pl.ds(row0, rpt)], a2_hbm.at[c, pl.ds(row0, rpt)])


def _gcn_aggregate_sc(x_cols, src_r, dst_r, zeros_tile, n_pad):
    fc = x_cols.shape[2]
    nchunks = src_r.shape[1]
    rpt = n_pad // NS
    mesh = plsc.VectorSubcoreMesh(core_axis_name="c", subcore_axis_name="s")
    body = functools.partial(_gcn_body, nchunks, rpt, fc)
    acc_t = jax.ShapeDtypeStruct((NC, n_pad, fc), jnp.float32)
    return pl.kernel(
        body,
        out_type=(acc_t, acc_t),
        mesh=mesh,
        compiler_params=pltpu.CompilerParams(use_tc_tiling_on_sc=False),
        scratch_types=[
            pltpu.VMEM((nchunks, CW), jnp.int32),
            pltpu.VMEM((nchunks, CW), jnp.int32),
            pltpu.VMEM((CW, fc), jnp.float32),
            pltpu.VMEM((CW, fc), jnp.float32),
            pltpu.VMEM_SHARED((n_pad, fc), jnp.float32),
            pltpu.SemaphoreType.DMA,
            pltpu.SemaphoreType.DMA,
        ],
    )(x_cols, src_r, dst_r, zeros_tile)


def _final_body(scale, fc, h0_ref, a1_ref, a2_ref, w_ref, b_ref, o_ref):
    w = w_ref[...]
    a1 = a1_ref[...]
    a2 = a2_ref[...]
    acc = jnp.dot(h0_ref[...], w[0], preferred_element_type=jnp.float32)
    acc += scale * (jnp.dot(a1[0], w[1, :fc], preferred_element_type=jnp.float32)
                    + jnp.dot(a1[1], w[1, fc:], preferred_element_type=jnp.float32))
    acc += (scale * scale) * (
        jnp.dot(a2[0], w[2, :fc], preferred_element_type=jnp.float32)
        + jnp.dot(a2[1], w[2, fc:], preferred_element_type=jnp.float32))
    o_ref[...] = acc + b_ref[...]


def kernel(h, edge_index, weight, bias):
    n, f = h.shape
    e = edge_index.shape[1]
    k = weight.shape[2]
    fc = f // NC
    assert k == 3 and e % (NS * CW) == 0 and f % NC == 0
    scale = 1.0 / n
    nchunks = e // (NS * CW)
    n_pad = ((n + NS * 8 - 1) // (NS * 8)) * NS * 8
    src_r = edge_index[0].reshape(NS, nchunks, CW)
    dst_r = edge_index[1].reshape(NS, nchunks, CW)
    x_cols = jnp.transpose(h.reshape(n, NC, fc), (1, 0, 2))
    x_cols = jnp.concatenate(
        [x_cols, jnp.zeros((NC, n_pad - n, fc), jnp.float32)], axis=1)
    zeros_tile = jnp.zeros((n_pad // NS, fc), jnp.float32)
    wt = jnp.transpose(weight, (2, 0, 1))

    a1, a2 = _gcn_aggregate_sc(x_cols, src_r, dst_r, zeros_tile, n_pad)

    bn = 1000
    grid = n // bn
    out = pl.pallas_call(
        functools.partial(_final_body, scale, fc),
        out_shape=jax.ShapeDtypeStruct((n, f), jnp.float32),
        grid=(grid,),
        in_specs=[
            pl.BlockSpec((bn, f), lambda i: (i, 0)),
            pl.BlockSpec((NC, bn, fc), lambda i: (0, i, 0)),
            pl.BlockSpec((NC, bn, fc), lambda i: (0, i, 0)),
            pl.BlockSpec((k, f, f), lambda i: (0, 0, 0)),
            pl.BlockSpec((1, f), lambda i: (0, 0)),
        ],
        out_specs=pl.BlockSpec((bn, f), lambda i: (i, 0)),
    )(h, a1, a2, wt, bias.reshape(1, f))
    return out
